# baseline (speedup 1.0000x reference)
import functools

import jax
import jax.numpy as jnp
from jax import lax
from jax.experimental import pallas as pl
from jax.experimental.pallas import tpu as pltpu
from jax.experimental.pallas import tpu_sc as plsc

N_NODES = 10000
IN_DIM = 128
HID_DIM = 128
NUM_CLASS = 47
N_EDGES = 320000

NC = 2
NS = 16
NW = NC * NS

C = 128
RC0 = 60
RC1 = 100
ROWS_PER_PAIR = RC0 + RC1
EP = ROWS_PER_PAIR * NS * C
NP = 10240
ZROWS = NP // NS
ZCH = ZROWS // C

_MESH = dict(core_axis_name="c", subcore_axis_name="s")


@functools.partial(
    pl.kernel,
    out_type=[jax.ShapeDtypeStruct((NC * NP, IN_DIM), jnp.float32)],
    mesh=plsc.VectorSubcoreMesh(**_MESH),
    scratch_types=[
        pltpu.VMEM((C, IN_DIM), jnp.float32),
        pltpu.VMEM_SHARED((NP, IN_DIM), jnp.float32),
        pltpu.VMEM((C,), jnp.int32),
        pltpu.VMEM((C,), jnp.int32),
        pltpu.SemaphoreType.DMA,
    ])
def _sc_agg(z, src, dst, table, out, rows, agg_sh, idx_s, idx_d, sem):
    c = lax.axis_index("c")
    s = lax.axis_index("s")
    nrows = jnp.where(c == 0, RC0, RC1)
    base0 = jnp.where(c == 0, s * RC0, NS * RC0 + s * RC1) * C

    pltpu.sync_copy(z, rows)
    for k in range(ZCH):
        pltpu.sync_copy(rows, agg_sh.at[pl.ds(s * ZROWS + k * C, C)])
    plsc.subcore_barrier()

    def body(i, carry):
        base = base0 + i * C
        pltpu.sync_copy(src.at[pl.ds(base, C)], idx_s)
        pltpu.sync_copy(dst.at[pl.ds(base, C)], idx_d)
        pltpu.async_copy(table.at[idx_s], rows, sem).wait()
        pltpu.sync_copy(rows, agg_sh.at[idx_d], add=True)
        return carry

    lax.fori_loop(0, nrows, body, 0)
    plsc.subcore_barrier()

    for k in range(ZCH):
        r0 = s * ZROWS + k * C
        pltpu.sync_copy(agg_sh.at[pl.ds(r0, C)], rows)
        pltpu.sync_copy(rows, out.at[pl.ds(c * NP + r0, C)])


@functools.partial(
    pl.kernel,
    out_type=[jax.ShapeDtypeStruct((NC * NP, IN_DIM), jnp.float32)],
    mesh=plsc.VectorSubcoreMesh(**_MESH),
    scratch_types=[
        pltpu.VMEM((C, IN_DIM), jnp.float32),
        pltpu.VMEM_SHARED((NP, IN_DIM), jnp.float32),
        pltpu.VMEM((C,), jnp.int32),
    ])
def _sc_deg(z, ones, dst, out, rows, deg_sh, idx_d):
    c = lax.axis_index("c")
    s = lax.axis_index("s")
    wid = s * NC + c
    base0 = wid * (EP // NW)

    pltpu.sync_copy(z, rows)
    for k in range(ZCH):
        pltpu.sync_copy(rows, deg_sh.at[pl.ds(s * ZROWS + k * C, C)])
    plsc.subcore_barrier()
    pltpu.sync_copy(ones, rows)

    def body(i, carry):
        base = base0 + i * C
        pltpu.sync_copy(dst.at[pl.ds(base, C)], idx_d)
        pltpu.sync_copy(rows, deg_sh.at[idx_d], add=True)
        return carry

    lax.fori_loop(0, EP // NW // C, body, 0)
    plsc.subcore_barrier()

    for k in range(ZCH):
        r0 = s * ZROWS + k * C
        pltpu.sync_copy(deg_sh.at[pl.ds(r0, C)], rows)
        pltpu.sync_copy(rows, out.at[pl.ds(c * NP + r0, C)])


def _tc_layer1(aggp, degp, x, W1l, b1l, W1r):
    R = 1000
    grid = (N_NODES // R,)

    def body(a_ref, d_ref, x_ref, wl_ref, b_ref, wr_ref, h_ref):
        a = a_ref[0] + a_ref[1]
        deg = jnp.maximum(d_ref[0, :, 0:1] + d_ref[1, :, 0:1], 1.0)
        mean = a / deg
        out = (lax.dot_general(mean, wl_ref[...], (((1,), (1,)), ((), ())),
                               preferred_element_type=jnp.float32,
                               precision=lax.Precision.HIGHEST)
               + lax.dot_general(x_ref[...], wr_ref[...],
                                 (((1,), (1,)), ((), ())),
                                 preferred_element_type=jnp.float32,
                                 precision=lax.Precision.HIGHEST)
               + b_ref[...])
        h_ref[...] = jnp.where(out >= 0, out, 0.01 * out)

    return pl.pallas_call(
        body,
        grid=grid,
        in_specs=[
            pl.BlockSpec((NC, R, IN_DIM), lambda i: (0, i, 0)),
            pl.BlockSpec((NC, R, IN_DIM), lambda i: (0, i, 0)),
            pl.BlockSpec((R, IN_DIM), lambda i: (i, 0)),
            pl.BlockSpec((HID_DIM, IN_DIM), lambda i: (0, 0)),
            pl.BlockSpec((1, HID_DIM), lambda i: (0, 0)),
            pl.BlockSpec((HID_DIM, IN_DIM), lambda i: (0, 0)),
        ],
        out_specs=pl.BlockSpec((R, HID_DIM), lambda i: (i, 0)),
        out_shape=jax.ShapeDtypeStruct((N_NODES, HID_DIM), jnp.float32),
    )(aggp, degp, x, W1l, b1l, W1r)


def _tc_layer2(aggp, degp, h, W2l, b2l, W2r):
    R = 1000
    grid = (N_NODES // R,)

    def body(a_ref, d_ref, h_ref, wl_ref, b_ref, wr_ref, o_ref):
        agg = a_ref[0] + a_ref[1]
        deg = jnp.maximum(d_ref[0, :, 0:1] + d_ref[1, :, 0:1], 1.0)
        mean = agg / deg
        logits = (lax.dot_general(mean, wl_ref[...], (((1,), (1,)), ((), ())),
                                  preferred_element_type=jnp.float32,
                                  precision=lax.Precision.HIGHEST)
                  + lax.dot_general(h_ref[...], wr_ref[...],
                                    (((1,), (1,)), ((), ())),
                                    preferred_element_type=jnp.float32,
                                    precision=lax.Precision.HIGHEST)
                  + b_ref[...])
        z = logits - jnp.max(logits, axis=1, keepdims=True)
        lse = jnp.log(jnp.sum(jnp.exp(z), axis=1, keepdims=True))
        o_ref[...] = z - lse

    return pl.pallas_call(
        body,
        grid=grid,
        in_specs=[
            pl.BlockSpec((NC, R, HID_DIM), lambda i: (0, i, 0)),
            pl.BlockSpec((NC, R, IN_DIM), lambda i: (0, i, 0)),
            pl.BlockSpec((R, HID_DIM), lambda i: (i, 0)),
            pl.BlockSpec((NUM_CLASS, HID_DIM), lambda i: (0, 0)),
            pl.BlockSpec((1, NUM_CLASS), lambda i: (0, 0)),
            pl.BlockSpec((NUM_CLASS, HID_DIM), lambda i: (0, 0)),
        ],
        out_specs=pl.BlockSpec((R, NUM_CLASS), lambda i: (i, 0)),
        out_shape=jax.ShapeDtypeStruct((N_NODES, NUM_CLASS), jnp.float32),
    )(aggp, degp, h, W2l, b2l, W2r)


def kernel(x, edge_index, W1l, b1l, W1r, W2l, b2l, W2r):
    src = edge_index[0].astype(jnp.int32)
    dst = edge_index[1].astype(jnp.int32)
    npad = EP - N_EDGES
    srcp = jnp.concatenate([src, jnp.zeros((npad,), jnp.int32)])
    dstp = jnp.concatenate([dst, jnp.full((npad,), N_NODES, jnp.int32)])

    z = jnp.zeros((C, IN_DIM), jnp.float32)
    ones = jnp.ones((C, IN_DIM), jnp.float32)

    (degp,) = _sc_deg(z, ones, dstp)
    (agg1,) = _sc_agg(z, srcp, dstp, x)
    degp = degp.reshape(NC, NP, IN_DIM)
    h = _tc_layer1(agg1.reshape(NC, NP, IN_DIM), degp, x, W1l,
                   b1l.reshape(1, -1), W1r)
    (agg2,) = _sc_agg(z, srcp, dstp, h)
    out = _tc_layer2(agg2.reshape(NC, NP, HID_DIM), degp, h, W2l,
                     b2l.reshape(1, -1), W2r)
    return out


# speedup vs baseline: 1.0793x; 1.0793x over previous
import functools

import jax
import jax.numpy as jnp
from jax import lax
from jax.experimental import pallas as pl
from jax.experimental.pallas import tpu as pltpu
from jax.experimental.pallas import tpu_sc as plsc

N_NODES = 10000
IN_DIM = 128
HID_DIM = 128
NUM_CLASS = 47
N_EDGES = 320000

NC = 2
NS = 16
NW = NC * NS

C = 128
ROWS_PER_W = 80
EP = ROWS_PER_W * NW * C
NP = 10240
ZROWS = NP // NS
ZCH = ZROWS // C

_MESH = dict(core_axis_name="c", subcore_axis_name="s")


@functools.partial(
    pl.kernel,
    out_type=[jax.ShapeDtypeStruct((NC * NP, IN_DIM), jnp.float32)],
    mesh=plsc.VectorSubcoreMesh(**_MESH),
    scratch_types=[
        pltpu.VMEM((C, IN_DIM), jnp.float32),
        pltpu.VMEM_SHARED((NP, IN_DIM), jnp.float32),
        pltpu.VMEM((C,), jnp.int32),
        pltpu.VMEM((C,), jnp.int32),
        pltpu.SemaphoreType.DMA,
    ])
def _sc_agg(z, src, dst, table, out, rows, agg_sh, idx_s, idx_d, sem):
    c = lax.axis_index("c")
    s = lax.axis_index("s")
    wid = s * NC + c
    base0 = wid * ROWS_PER_W * C

    pltpu.sync_copy(z, rows)
    for k in range(ZCH):
        pltpu.sync_copy(rows, agg_sh.at[pl.ds(s * ZROWS + k * C, C)])
    plsc.subcore_barrier()

    def body(i, carry):
        base = base0 + i * C
        pltpu.sync_copy(src.at[pl.ds(base, C)], idx_s)
        pltpu.sync_copy(dst.at[pl.ds(base, C)], idx_d)
        pltpu.async_copy(table.at[idx_s], rows, sem).wait()
        pltpu.sync_copy(rows, agg_sh.at[idx_d], add=True)
        return carry

    lax.fori_loop(0, ROWS_PER_W, body, 0)
    plsc.subcore_barrier()

    for k in range(ZCH):
        r0 = s * ZROWS + k * C
        pltpu.sync_copy(agg_sh.at[pl.ds(r0, C)], rows)
        pltpu.sync_copy(rows, out.at[pl.ds(c * NP + r0, C)])


@functools.partial(
    pl.kernel,
    out_type=[jax.ShapeDtypeStruct((NC * NP, IN_DIM), jnp.float32)],
    mesh=plsc.VectorSubcoreMesh(**_MESH),
    scratch_types=[
        pltpu.VMEM((C, IN_DIM), jnp.float32),
        pltpu.VMEM_SHARED((NP, IN_DIM), jnp.float32),
        pltpu.VMEM((C,), jnp.int32),
    ])
def _sc_deg(z, ones, dst, out, rows, deg_sh, idx_d):
    c = lax.axis_index("c")
    s = lax.axis_index("s")
    wid = s * NC + c
    base0 = wid * (EP // NW)

    pltpu.sync_copy(z, rows)
    for k in range(ZCH):
        pltpu.sync_copy(rows, deg_sh.at[pl.ds(s * ZROWS + k * C, C)])
    plsc.subcore_barrier()
    pltpu.sync_copy(ones, rows)

    def body(i, carry):
        base = base0 + i * C
        pltpu.sync_copy(dst.at[pl.ds(base, C)], idx_d)
        pltpu.sync_copy(rows, deg_sh.at[idx_d], add=True)
        return carry

    lax.fori_loop(0, EP // NW // C, body, 0)
    plsc.subcore_barrier()

    for k in range(ZCH):
        r0 = s * ZROWS + k * C
        pltpu.sync_copy(deg_sh.at[pl.ds(r0, C)], rows)
        pltpu.sync_copy(rows, out.at[pl.ds(c * NP + r0, C)])


def _tc_layer1(aggp, degp, x, W1l, b1l, W1r):
    R = 1000
    grid = (N_NODES // R,)

    def body(a_ref, d_ref, x_ref, wl_ref, b_ref, wr_ref, h_ref):
        a = a_ref[0] + a_ref[1]
        deg = jnp.maximum(d_ref[0, :, 0:1] + d_ref[1, :, 0:1], 1.0)
        mean = a / deg
        out = (lax.dot_general(mean, wl_ref[...], (((1,), (1,)), ((), ())),
                               preferred_element_type=jnp.float32,
                               precision=lax.Precision.HIGHEST)
               + lax.dot_general(x_ref[...], wr_ref[...],
                                 (((1,), (1,)), ((), ())),
                                 preferred_element_type=jnp.float32,
                                 precision=lax.Precision.HIGHEST)
               + b_ref[...])
        h_ref[...] = jnp.where(out >= 0, out, 0.01 * out)

    return pl.pallas_call(
        body,
        grid=grid,
        in_specs=[
            pl.BlockSpec((NC, R, IN_DIM), lambda i: (0, i, 0)),
            pl.BlockSpec((NC, R, IN_DIM), lambda i: (0, i, 0)),
            pl.BlockSpec((R, IN_DIM), lambda i: (i, 0)),
            pl.BlockSpec((HID_DIM, IN_DIM), lambda i: (0, 0)),
            pl.BlockSpec((1, HID_DIM), lambda i: (0, 0)),
            pl.BlockSpec((HID_DIM, IN_DIM), lambda i: (0, 0)),
        ],
        out_specs=pl.BlockSpec((R, HID_DIM), lambda i: (i, 0)),
        out_shape=jax.ShapeDtypeStruct((N_NODES, HID_DIM), jnp.float32),
    )(aggp, degp, x, W1l, b1l, W1r)


def _tc_layer2(aggp, degp, h, W2l, b2l, W2r):
    R = 1000
    grid = (N_NODES // R,)

    def body(a_ref, d_ref, h_ref, wl_ref, b_ref, wr_ref, o_ref):
        agg = a_ref[0] + a_ref[1]
        deg = jnp.maximum(d_ref[0, :, 0:1] + d_ref[1, :, 0:1], 1.0)
        mean = agg / deg
        logits = (lax.dot_general(mean, wl_ref[...], (((1,), (1,)), ((), ())),
                                  preferred_element_type=jnp.float32,
                                  precision=lax.Precision.HIGHEST)
                  + lax.dot_general(h_ref[...], wr_ref[...],
                                    (((1,), (1,)), ((), ())),
                                    preferred_element_type=jnp.float32,
                                    precision=lax.Precision.HIGHEST)
                  + b_ref[...])
        z = logits - jnp.max(logits, axis=1, keepdims=True)
        lse = jnp.log(jnp.sum(jnp.exp(z), axis=1, keepdims=True))
        o_ref[...] = z - lse

    return pl.pallas_call(
        body,
        grid=grid,
        in_specs=[
            pl.BlockSpec((NC, R, HID_DIM), lambda i: (0, i, 0)),
            pl.BlockSpec((NC, R, IN_DIM), lambda i: (0, i, 0)),
            pl.BlockSpec((R, HID_DIM), lambda i: (i, 0)),
            pl.BlockSpec((NUM_CLASS, HID_DIM), lambda i: (0, 0)),
            pl.BlockSpec((1, NUM_CLASS), lambda i: (0, 0)),
            pl.BlockSpec((NUM_CLASS, HID_DIM), lambda i: (0, 0)),
        ],
        out_specs=pl.BlockSpec((R, NUM_CLASS), lambda i: (i, 0)),
        out_shape=jax.ShapeDtypeStruct((N_NODES, NUM_CLASS), jnp.float32),
    )(aggp, degp, h, W2l, b2l, W2r)


def kernel(x, edge_index, W1l, b1l, W1r, W2l, b2l, W2r):
    src = edge_index[0].astype(jnp.int32)
    dst = edge_index[1].astype(jnp.int32)
    npad = EP - N_EDGES
    srcp = jnp.concatenate([src, jnp.zeros((npad,), jnp.int32)])
    dstp = jnp.concatenate([dst, jnp.full((npad,), N_NODES, jnp.int32)])

    z = jnp.zeros((C, IN_DIM), jnp.float32)
    ones = jnp.ones((C, IN_DIM), jnp.float32)

    (degp,) = _sc_deg(z, ones, dstp)
    (agg1,) = _sc_agg(z, srcp, dstp, x)
    degp = degp.reshape(NC, NP, IN_DIM)
    h = _tc_layer1(agg1.reshape(NC, NP, IN_DIM), degp, x, W1l,
                   b1l.reshape(1, -1), W1r)
    (agg2,) = _sc_agg(z, srcp, dstp, h)
    out = _tc_layer2(agg2.reshape(NC, NP, HID_DIM), degp, h, W2l,
                     b2l.reshape(1, -1), W2r)
    return out


# speedup vs baseline: 1.0803x; 1.0009x over previous
import functools

import jax
import jax.numpy as jnp
from jax import lax
from jax.experimental import pallas as pl
from jax.experimental.pallas import tpu as pltpu
from jax.experimental.pallas import tpu_sc as plsc

N_NODES = 10000
IN_DIM = 128
HID_DIM = 128
NUM_CLASS = 47
N_EDGES = 320000

NC = 2
NS = 16
NW = NC * NS

C = 128
ROWS_PER_W = 80
EP = ROWS_PER_W * NW * C
NP = 10240
ZROWS = NP // NS
ZCH = ZROWS // C

_MESH = dict(core_axis_name="c", subcore_axis_name="s")


@functools.partial(
    pl.kernel,
    out_type=[jax.ShapeDtypeStruct((NC * NP, IN_DIM), jnp.float32)],
    mesh=plsc.VectorSubcoreMesh(**_MESH),
    scratch_types=[
        pltpu.VMEM((C, IN_DIM), jnp.float32),
        pltpu.VMEM_SHARED((NP, IN_DIM), jnp.float32),
        pltpu.VMEM((C,), jnp.int32),
        pltpu.VMEM((C,), jnp.int32),
        pltpu.SemaphoreType.DMA,
    ])
def _sc_agg(z, src, dst, table, out, rows, agg_sh, idx_s, idx_d, sem):
    c = lax.axis_index("c")
    s = lax.axis_index("s")
    wid = s * NC + c
    base0 = wid * ROWS_PER_W * C

    pltpu.sync_copy(z, rows)
    for k in range(ZCH):
        pltpu.sync_copy(rows, agg_sh.at[pl.ds(s * ZROWS + k * C, C)])
    plsc.subcore_barrier()

    def body(i, carry):
        base = base0 + i * C
        pltpu.sync_copy(src.at[pl.ds(base, C)], idx_s)
        pltpu.sync_copy(dst.at[pl.ds(base, C)], idx_d)
        pltpu.async_copy(table.at[idx_s], rows, sem).wait()
        pltpu.sync_copy(rows, agg_sh.at[idx_d], add=True)
        return carry

    lax.fori_loop(0, ROWS_PER_W, body, 0)
    plsc.subcore_barrier()

    for k in range(ZCH):
        r0 = s * ZROWS + k * C
        pltpu.sync_copy(agg_sh.at[pl.ds(r0, C)], rows)
        pltpu.sync_copy(rows, out.at[pl.ds(c * NP + r0, C)])


@functools.partial(
    pl.kernel,
    out_type=[jax.ShapeDtypeStruct((NC * NP, IN_DIM), jnp.float32)],
    mesh=plsc.VectorSubcoreMesh(**_MESH),
    scratch_types=[
        pltpu.VMEM((C, IN_DIM), jnp.float32),
        pltpu.VMEM_SHARED((NP, IN_DIM), jnp.float32),
        pltpu.VMEM((C,), jnp.int32),
    ])
def _sc_deg(z, ones, dst, out, rows, deg_sh, idx_d):
    c = lax.axis_index("c")
    s = lax.axis_index("s")
    wid = s * NC + c
    base0 = wid * (EP // NW)

    pltpu.sync_copy(z, rows)
    for k in range(ZCH):
        pltpu.sync_copy(rows, deg_sh.at[pl.ds(s * ZROWS + k * C, C)])
    plsc.subcore_barrier()
    pltpu.sync_copy(ones, rows)

    def body(i, carry):
        base = base0 + i * C
        pltpu.sync_copy(dst.at[pl.ds(base, C)], idx_d)
        pltpu.sync_copy(rows, deg_sh.at[idx_d], add=True)
        return carry

    lax.fori_loop(0, EP // NW // C, body, 0)
    plsc.subcore_barrier()

    for k in range(ZCH):
        r0 = s * ZROWS + k * C
        pltpu.sync_copy(deg_sh.at[pl.ds(r0, C)], rows)
        pltpu.sync_copy(rows, out.at[pl.ds(c * NP + r0, C)])


def _tc_layer1(aggp, degp, x, W1l, b1l, W1r):
    R = 1000
    grid = (N_NODES // R,)

    def body(a_ref, d_ref, x_ref, wl_ref, b_ref, wr_ref, h_ref):
        a = a_ref[0] + a_ref[1]
        deg = jnp.maximum(d_ref[0, :, 0:1] + d_ref[1, :, 0:1], 1.0)
        mean = a / deg
        out = (lax.dot_general(mean, wl_ref[...], (((1,), (1,)), ((), ())),
                               preferred_element_type=jnp.float32,
                               precision=lax.Precision.HIGHEST)
               + lax.dot_general(x_ref[...], wr_ref[...],
                                 (((1,), (1,)), ((), ())),
                                 preferred_element_type=jnp.float32,
                                 precision=lax.Precision.HIGHEST)
               + b_ref[...])
        h_ref[...] = jnp.where(out >= 0, out, 0.01 * out)

    return pl.pallas_call(
        body,
        grid=grid,
        in_specs=[
            pl.BlockSpec((NC, R, IN_DIM), lambda i: (0, i, 0)),
            pl.BlockSpec((NC, R, IN_DIM), lambda i: (0, i, 0)),
            pl.BlockSpec((R, IN_DIM), lambda i: (i, 0)),
            pl.BlockSpec((HID_DIM, IN_DIM), lambda i: (0, 0)),
            pl.BlockSpec((1, HID_DIM), lambda i: (0, 0)),
            pl.BlockSpec((HID_DIM, IN_DIM), lambda i: (0, 0)),
        ],
        out_specs=pl.BlockSpec((R, HID_DIM), lambda i: (i, 0)),
        out_shape=jax.ShapeDtypeStruct((N_NODES, HID_DIM), jnp.float32),
    )(aggp, degp, x, W1l, b1l, W1r)


def _tc_layer2(aggp, degp, h, W2l, b2l, W2r):
    R = 1000
    grid = (N_NODES // R,)

    def body(a_ref, d_ref, h_ref, wl_ref, b_ref, wr_ref, o_ref):
        agg = a_ref[0] + a_ref[1]
        deg = jnp.maximum(d_ref[0, :, 0:1] + d_ref[1, :, 0:1], 1.0)
        mean = agg / deg
        logits = (lax.dot_general(mean, wl_ref[...], (((1,), (1,)), ((), ())),
                                  preferred_element_type=jnp.float32,
                                  precision=lax.Precision.HIGHEST)
                  + lax.dot_general(h_ref[...], wr_ref[...],
                                    (((1,), (1,)), ((), ())),
                                    preferred_element_type=jnp.float32,
                                    precision=lax.Precision.HIGHEST)
                  + b_ref[...])
        z = logits - jnp.max(logits, axis=1, keepdims=True)
        lse = jnp.log(jnp.sum(jnp.exp(z), axis=1, keepdims=True))
        o_ref[...] = z - lse

    return pl.pallas_call(
        body,
        grid=grid,
        in_specs=[
            pl.BlockSpec((NC, R, HID_DIM), lambda i: (0, i, 0)),
            pl.BlockSpec((NC, R, IN_DIM), lambda i: (0, i, 0)),
            pl.BlockSpec((R, HID_DIM), lambda i: (i, 0)),
            pl.BlockSpec((NUM_CLASS, HID_DIM), lambda i: (0, 0)),
            pl.BlockSpec((1, NUM_CLASS), lambda i: (0, 0)),
            pl.BlockSpec((NUM_CLASS, HID_DIM), lambda i: (0, 0)),
        ],
        out_specs=pl.BlockSpec((R, NUM_CLASS), lambda i: (i, 0)),
        out_shape=jax.ShapeDtypeStruct((N_NODES, NUM_CLASS), jnp.float32),
    )(aggp, degp, h, W2l, b2l, W2r)


def kernel(x, edge_index, W1l, b1l, W1r, W2l, b2l, W2r):
    src = edge_index[0].astype(jnp.int32)
    dst = edge_index[1].astype(jnp.int32)
    npad = EP - N_EDGES
    srcp = jnp.concatenate([src, jnp.zeros((npad,), jnp.int32)])
    padd = N_NODES + jnp.arange(npad, dtype=jnp.int32) % (NP - N_NODES)
    dstp = jnp.concatenate([dst, padd])

    z = jnp.zeros((C, IN_DIM), jnp.float32)
    ones = jnp.ones((C, IN_DIM), jnp.float32)

    (degp,) = _sc_deg(z, ones, dstp)
    (agg1,) = _sc_agg(z, srcp, dstp, x)
    degp = degp.reshape(NC, NP, IN_DIM)
    h = _tc_layer1(agg1.reshape(NC, NP, IN_DIM), degp, x, W1l,
                   b1l.reshape(1, -1), W1r)
    (agg2,) = _sc_agg(z, srcp, dstp, h)
    out = _tc_layer2(agg2.reshape(NC, NP, HID_DIM), degp, h, W2l,
                     b2l.reshape(1, -1), W2r)
    return out


# speedup vs baseline: 2.0089x; 1.8596x over previous
import functools

import jax
import jax.numpy as jnp
from jax import lax
from jax.experimental import pallas as pl
from jax.experimental.pallas import tpu as pltpu
from jax.experimental.pallas import tpu_sc as plsc

N_NODES = 10000
IN_DIM = 128
HID_DIM = 128
NUM_CLASS = 47
N_EDGES = 320000

NC = 2
NS = 16
NW = NC * NS

C = 128
ROWS_PER_W = 80
EP = ROWS_PER_W * NW * C
NP = 10240
ZROWS = NP // NS
ZCH = ZROWS // C

_MESH = dict(core_axis_name="c", subcore_axis_name="s")


@functools.partial(
    pl.kernel,
    out_type=[jax.ShapeDtypeStruct((NC * NP, IN_DIM), jnp.float32)],
    mesh=plsc.VectorSubcoreMesh(**_MESH),
    scratch_types=[
        pltpu.VMEM((C, IN_DIM), jnp.float32),
        pltpu.VMEM_SHARED((NP, IN_DIM), jnp.float32),
        pltpu.VMEM((C,), jnp.int32),
        pltpu.VMEM((C,), jnp.int32),
        pltpu.SemaphoreType.DMA,
    ])
def _sc_agg(z, src, dst, table, out, rows, agg_sh, idx_s, idx_d, sem):
    c = lax.axis_index("c")
    s = lax.axis_index("s")
    wid = s * NC + c
    base0 = wid * ROWS_PER_W * C

    pltpu.sync_copy(z, rows)
    for k in range(ZCH):
        pltpu.sync_copy(rows, agg_sh.at[pl.ds(s * ZROWS + k * C, C)])
    plsc.subcore_barrier()

    def body(i, carry):
        base = base0 + i * C
        pltpu.sync_copy(src.at[pl.ds(base, C)], idx_s)
        pltpu.sync_copy(dst.at[pl.ds(base, C)], idx_d)
        pltpu.async_copy(table.at[idx_s], rows, sem).wait()
        pltpu.sync_copy(rows, agg_sh.at[idx_d], add=True)
        return carry

    lax.fori_loop(0, ROWS_PER_W, body, 0)
    plsc.subcore_barrier()

    for k in range(ZCH):
        r0 = s * ZROWS + k * C
        pltpu.sync_copy(agg_sh.at[pl.ds(r0, C)], rows)
        pltpu.sync_copy(rows, out.at[pl.ds(c * NP + r0, C)])


@functools.partial(
    pl.kernel,
    out_type=[jax.ShapeDtypeStruct((NC * NP, IN_DIM), jnp.float32)],
    mesh=plsc.VectorSubcoreMesh(**_MESH),
    scratch_types=[
        pltpu.VMEM((C, IN_DIM), jnp.float32),
        pltpu.VMEM_SHARED((NP, IN_DIM), jnp.float32),
        pltpu.VMEM((C,), jnp.int32),
    ])
def _sc_deg(z, ones, dst, out, rows, deg_sh, idx_d):
    c = lax.axis_index("c")
    s = lax.axis_index("s")
    wid = s * NC + c
    base0 = wid * (EP // NW)

    pltpu.sync_copy(z, rows)
    for k in range(ZCH):
        pltpu.sync_copy(rows, deg_sh.at[pl.ds(s * ZROWS + k * C, C)])
    plsc.subcore_barrier()
    pltpu.sync_copy(ones, rows)

    def body(i, carry):
        base = base0 + i * C
        pltpu.sync_copy(dst.at[pl.ds(base, C)], idx_d)
        pltpu.sync_copy(rows, deg_sh.at[idx_d], add=True)
        return carry

    lax.fori_loop(0, EP // NW // C, body, 0)
    plsc.subcore_barrier()

    for k in range(ZCH):
        r0 = s * ZROWS + k * C
        pltpu.sync_copy(deg_sh.at[pl.ds(r0, C)], rows)
        pltpu.sync_copy(rows, out.at[pl.ds(c * NP + r0, C)])


def _tc_layer1(aggp, degp, x, W1l, b1l, W1r):
    R = 1000
    grid = (N_NODES // R,)

    def body(a_ref, d_ref, x_ref, wl_ref, b_ref, wr_ref, h_ref):
        a = a_ref[0] + a_ref[1]
        deg = jnp.maximum(d_ref[0, :, 0:1] + d_ref[1, :, 0:1], 1.0)
        mean = a / deg
        out = (lax.dot_general(mean, wl_ref[...], (((1,), (1,)), ((), ())),
                               preferred_element_type=jnp.float32,
                               precision=lax.Precision.HIGHEST)
               + lax.dot_general(x_ref[...], wr_ref[...],
                                 (((1,), (1,)), ((), ())),
                                 preferred_element_type=jnp.float32,
                                 precision=lax.Precision.HIGHEST)
               + b_ref[...])
        h_ref[...] = jnp.where(out >= 0, out, 0.01 * out)

    return pl.pallas_call(
        body,
        grid=grid,
        in_specs=[
            pl.BlockSpec((NC, R, IN_DIM), lambda i: (0, i, 0)),
            pl.BlockSpec((NC, R, IN_DIM), lambda i: (0, i, 0)),
            pl.BlockSpec((R, IN_DIM), lambda i: (i, 0)),
            pl.BlockSpec((HID_DIM, IN_DIM), lambda i: (0, 0)),
            pl.BlockSpec((1, HID_DIM), lambda i: (0, 0)),
            pl.BlockSpec((HID_DIM, IN_DIM), lambda i: (0, 0)),
        ],
        out_specs=pl.BlockSpec((R, HID_DIM), lambda i: (i, 0)),
        out_shape=jax.ShapeDtypeStruct((N_NODES, HID_DIM), jnp.float32),
    )(aggp, degp, x, W1l, b1l, W1r)


def _tc_layer2(aggp, degp, h, W2l, b2l, W2r):
    R = 1000
    grid = (N_NODES // R,)

    def body(a_ref, d_ref, h_ref, wl_ref, b_ref, wr_ref, o_ref):
        agg = a_ref[0] + a_ref[1]
        deg = jnp.maximum(d_ref[0, :, 0:1] + d_ref[1, :, 0:1], 1.0)
        mean = agg / deg
        logits = (lax.dot_general(mean, wl_ref[...], (((1,), (1,)), ((), ())),
                                  preferred_element_type=jnp.float32,
                                  precision=lax.Precision.HIGHEST)
                  + lax.dot_general(h_ref[...], wr_ref[...],
                                    (((1,), (1,)), ((), ())),
                                    preferred_element_type=jnp.float32,
                                    precision=lax.Precision.HIGHEST)
                  + b_ref[...])
        z = logits - jnp.max(logits, axis=1, keepdims=True)
        lse = jnp.log(jnp.sum(jnp.exp(z), axis=1, keepdims=True))
        o_ref[...] = z - lse

    return pl.pallas_call(
        body,
        grid=grid,
        in_specs=[
            pl.BlockSpec((NC, R, HID_DIM), lambda i: (0, i, 0)),
            pl.BlockSpec((NC, R, IN_DIM), lambda i: (0, i, 0)),
            pl.BlockSpec((R, HID_DIM), lambda i: (i, 0)),
            pl.BlockSpec((NUM_CLASS, HID_DIM), lambda i: (0, 0)),
            pl.BlockSpec((1, NUM_CLASS), lambda i: (0, 0)),
            pl.BlockSpec((NUM_CLASS, HID_DIM), lambda i: (0, 0)),
        ],
        out_specs=pl.BlockSpec((R, NUM_CLASS), lambda i: (i, 0)),
        out_shape=jax.ShapeDtypeStruct((N_NODES, NUM_CLASS), jnp.float32),
    )(aggp, degp, h, W2l, b2l, W2r)


def kernel(x, edge_index, W1l, b1l, W1r, W2l, b2l, W2r):
    src = edge_index[0].astype(jnp.int32)
    dst = edge_index[1].astype(jnp.int32)
    npad = EP - N_EDGES
    pads = jnp.arange(npad, dtype=jnp.int32) * 131 % N_NODES
    srcp = jnp.concatenate([src, pads])
    padd = N_NODES + jnp.arange(npad, dtype=jnp.int32) % (NP - N_NODES)
    dstp = jnp.concatenate([dst, padd])

    z = jnp.zeros((C, IN_DIM), jnp.float32)
    ones = jnp.ones((C, IN_DIM), jnp.float32)

    (degp,) = _sc_deg(z, ones, dstp)
    (agg1,) = _sc_agg(z, srcp, dstp, x)
    degp = degp.reshape(NC, NP, IN_DIM)
    h = _tc_layer1(agg1.reshape(NC, NP, IN_DIM), degp, x, W1l,
                   b1l.reshape(1, -1), W1r)
    (agg2,) = _sc_agg(z, srcp, dstp, h)
    out = _tc_layer2(agg2.reshape(NC, NP, HID_DIM), degp, h, W2l,
                     b2l.reshape(1, -1), W2r)
    return out


# speedup vs baseline: 2.8212x; 1.4044x over previous
import functools

import jax
import jax.numpy as jnp
from jax import lax
from jax.experimental import pallas as pl
from jax.experimental.pallas import tpu as pltpu
from jax.experimental.pallas import tpu_sc as plsc

N_NODES = 10000
IN_DIM = 128
HID_DIM = 128
NUM_CLASS = 47
N_EDGES = 320000

NC = 2
NS = 16
NW = NC * NS

C = 128
ROWS_PER_W = 80
NPAIR = ROWS_PER_W // 2
EP = ROWS_PER_W * NW * C
NP = 10240
ZROWS = NP // NS
ZCH = ZROWS // C

_MESH = dict(core_axis_name="c", subcore_axis_name="s")


@functools.partial(
    pl.kernel,
    out_type=[jax.ShapeDtypeStruct((NC * NP, IN_DIM), jnp.float32)],
    mesh=plsc.VectorSubcoreMesh(**_MESH),
    scratch_types=[
        pltpu.VMEM((C, IN_DIM), jnp.float32),
        pltpu.VMEM((C, IN_DIM), jnp.float32),
        pltpu.VMEM_SHARED((NP, IN_DIM), jnp.float32),
        pltpu.VMEM((C,), jnp.int32),
        pltpu.VMEM((C,), jnp.int32),
        pltpu.VMEM((C,), jnp.int32),
        pltpu.VMEM((C,), jnp.int32),
        pltpu.SemaphoreType.DMA,
        pltpu.SemaphoreType.DMA,
    ])
def _sc_agg(z, src, dst, table, out, rows, rows_b, agg_sh,
            idx_s, idx_d, idxs_b, idxd_b, sem, sem_b):
    c = lax.axis_index("c")
    s = lax.axis_index("s")
    wid = s * NC + c
    base0 = wid * ROWS_PER_W * C

    pltpu.sync_copy(z, rows)
    for k in range(ZCH):
        pltpu.sync_copy(rows, agg_sh.at[pl.ds(s * ZROWS + k * C, C)])
    plsc.subcore_barrier()

    pltpu.sync_copy(src.at[pl.ds(base0, C)], idx_s)
    pltpu.sync_copy(dst.at[pl.ds(base0, C)], idx_d)
    pltpu.async_copy(table.at[idx_s], rows, sem)

    def body(i2, carry):
        cb = base0 + (2 * i2 + 1) * C
        cn = cb + C
        pltpu.sync_copy(src.at[pl.ds(cb, C)], idxs_b)
        pltpu.sync_copy(dst.at[pl.ds(cb, C)], idxd_b)
        pltpu.async_copy(table.at[idxs_b], rows_b, sem_b)
        pltpu.make_async_copy(table.at[idx_s], rows, sem).wait()
        pltpu.sync_copy(rows, agg_sh.at[idx_d], add=True)
        @pl.when(i2 < NPAIR - 1)
        def _():
            pltpu.sync_copy(src.at[pl.ds(cn, C)], idx_s)
            pltpu.sync_copy(dst.at[pl.ds(cn, C)], idx_d)
            pltpu.async_copy(table.at[idx_s], rows, sem)
        pltpu.make_async_copy(table.at[idxs_b], rows_b, sem_b).wait()
        pltpu.sync_copy(rows_b, agg_sh.at[idxd_b], add=True)
        return carry

    lax.fori_loop(0, NPAIR, body, 0)
    plsc.subcore_barrier()

    for k in range(ZCH):
        r0 = s * ZROWS + k * C
        pltpu.sync_copy(agg_sh.at[pl.ds(r0, C)], rows)
        pltpu.sync_copy(rows, out.at[pl.ds(c * NP + r0, C)])


@functools.partial(
    pl.kernel,
    out_type=[jax.ShapeDtypeStruct((NC * NP, IN_DIM), jnp.float32)],
    mesh=plsc.VectorSubcoreMesh(**_MESH),
    scratch_types=[
        pltpu.VMEM((C, IN_DIM), jnp.float32),
        pltpu.VMEM_SHARED((NP, IN_DIM), jnp.float32),
        pltpu.VMEM((C,), jnp.int32),
    ])
def _sc_deg(z, ones, dst, out, rows, deg_sh, idx_d):
    c = lax.axis_index("c")
    s = lax.axis_index("s")
    wid = s * NC + c
    base0 = wid * (EP // NW)

    pltpu.sync_copy(z, rows)
    for k in range(ZCH):
        pltpu.sync_copy(rows, deg_sh.at[pl.ds(s * ZROWS + k * C, C)])
    plsc.subcore_barrier()
    pltpu.sync_copy(ones, rows)

    def body(i, carry):
        base = base0 + i * C
        pltpu.sync_copy(dst.at[pl.ds(base, C)], idx_d)
        pltpu.sync_copy(rows, deg_sh.at[idx_d], add=True)
        return carry

    lax.fori_loop(0, EP // NW // C, body, 0)
    plsc.subcore_barrier()

    for k in range(ZCH):
        r0 = s * ZROWS + k * C
        pltpu.sync_copy(deg_sh.at[pl.ds(r0, C)], rows)
        pltpu.sync_copy(rows, out.at[pl.ds(c * NP + r0, C)])


def _tc_layer1(aggp, degp, x, W1l, b1l, W1r):
    R = 1000
    grid = (N_NODES // R,)

    def body(a_ref, d_ref, x_ref, wl_ref, b_ref, wr_ref, h_ref):
        a = a_ref[0] + a_ref[1]
        deg = jnp.maximum(d_ref[0, :, 0:1] + d_ref[1, :, 0:1], 1.0)
        mean = a / deg
        out = (lax.dot_general(mean, wl_ref[...], (((1,), (1,)), ((), ())),
                               preferred_element_type=jnp.float32,
                               precision=lax.Precision.HIGHEST)
               + lax.dot_general(x_ref[...], wr_ref[...],
                                 (((1,), (1,)), ((), ())),
                                 preferred_element_type=jnp.float32,
                                 precision=lax.Precision.HIGHEST)
               + b_ref[...])
        h_ref[...] = jnp.where(out >= 0, out, 0.01 * out)

    return pl.pallas_call(
        body,
        grid=grid,
        in_specs=[
            pl.BlockSpec((NC, R, IN_DIM), lambda i: (0, i, 0)),
            pl.BlockSpec((NC, R, IN_DIM), lambda i: (0, i, 0)),
            pl.BlockSpec((R, IN_DIM), lambda i: (i, 0)),
            pl.BlockSpec((HID_DIM, IN_DIM), lambda i: (0, 0)),
            pl.BlockSpec((1, HID_DIM), lambda i: (0, 0)),
            pl.BlockSpec((HID_DIM, IN_DIM), lambda i: (0, 0)),
        ],
        out_specs=pl.BlockSpec((R, HID_DIM), lambda i: (i, 0)),
        out_shape=jax.ShapeDtypeStruct((N_NODES, HID_DIM), jnp.float32),
    )(aggp, degp, x, W1l, b1l, W1r)


def _tc_layer2(aggp, degp, h, W2l, b2l, W2r):
    R = 1000
    grid = (N_NODES // R,)

    def body(a_ref, d_ref, h_ref, wl_ref, b_ref, wr_ref, o_ref):
        agg = a_ref[0] + a_ref[1]
        deg = jnp.maximum(d_ref[0, :, 0:1] + d_ref[1, :, 0:1], 1.0)
        mean = agg / deg
        logits = (lax.dot_general(mean, wl_ref[...], (((1,), (1,)), ((), ())),
                                  preferred_element_type=jnp.float32,
                                  precision=lax.Precision.HIGHEST)
                  + lax.dot_general(h_ref[...], wr_ref[...],
                                    (((1,), (1,)), ((), ())),
                                    preferred_element_type=jnp.float32,
                                    precision=lax.Precision.HIGHEST)
                  + b_ref[...])
        z = logits - jnp.max(logits, axis=1, keepdims=True)
        lse = jnp.log(jnp.sum(jnp.exp(z), axis=1, keepdims=True))
        o_ref[...] = z - lse

    return pl.pallas_call(
        body,
        grid=grid,
        in_specs=[
            pl.BlockSpec((NC, R, HID_DIM), lambda i: (0, i, 0)),
            pl.BlockSpec((NC, R, IN_DIM), lambda i: (0, i, 0)),
            pl.BlockSpec((R, HID_DIM), lambda i: (i, 0)),
            pl.BlockSpec((NUM_CLASS, HID_DIM), lambda i: (0, 0)),
            pl.BlockSpec((1, NUM_CLASS), lambda i: (0, 0)),
            pl.BlockSpec((NUM_CLASS, HID_DIM), lambda i: (0, 0)),
        ],
        out_specs=pl.BlockSpec((R, NUM_CLASS), lambda i: (i, 0)),
        out_shape=jax.ShapeDtypeStruct((N_NODES, NUM_CLASS), jnp.float32),
    )(aggp, degp, h, W2l, b2l, W2r)


def kernel(x, edge_index, W1l, b1l, W1r, W2l, b2l, W2r):
    src = edge_index[0].astype(jnp.int32)
    dst = edge_index[1].astype(jnp.int32)
    npad = EP - N_EDGES
    pads = jnp.arange(npad, dtype=jnp.int32) * 131 % N_NODES
    srcp = jnp.concatenate([src, pads])
    padd = N_NODES + jnp.arange(npad, dtype=jnp.int32) % (NP - N_NODES)
    dstp = jnp.concatenate([dst, padd])

    z = jnp.zeros((C, IN_DIM), jnp.float32)
    ones = jnp.ones((C, IN_DIM), jnp.float32)

    (degp,) = _sc_deg(z, ones, dstp)
    (agg1,) = _sc_agg(z, srcp, dstp, x)
    degp = degp.reshape(NC, NP, IN_DIM)
    h = _tc_layer1(agg1.reshape(NC, NP, IN_DIM), degp, x, W1l,
                   b1l.reshape(1, -1), W1r)
    (agg2,) = _sc_agg(z, srcp, dstp, h)
    out = _tc_layer2(agg2.reshape(NC, NP, HID_DIM), degp, h, W2l,
                     b2l.reshape(1, -1), W2r)
    return out
